# Initial kernel scaffold; baseline (speedup 1.0000x reference)
import jax
import jax.numpy as jnp
from jax.experimental import pallas as pl


def kernel(x, Wg, W1, W2, W3):
    raise NotImplementedError("write your pallas kernel here")


# speedup vs baseline: 1.8663x; 1.8663x over previous
import jax
import jax.numpy as jnp
from jax.experimental import pallas as pl
from jax.experimental.pallas import tpu as pltpu

NUM_EXPERTS = 8
TOP_K = 2


def _moe_dense_kernel(x_ref, wg_ref, w1_ref, w2_ref, w3_ref, out_ref):
    e = pl.program_id(0)
    xs = x_ref[...]

    scores = jnp.dot(xs, wg_ref[...], preferred_element_type=jnp.float32)
    E = scores.shape[-1]
    iota = jax.lax.broadcasted_iota(jnp.int32, scores.shape, 1)
    m1 = jnp.max(scores, axis=-1, keepdims=True)
    idx1 = jnp.min(jnp.where(scores == m1, iota, E), axis=-1, keepdims=True)
    oh1 = iota == idx1
    scores2 = jnp.where(oh1, -jnp.inf, scores)
    m2 = jnp.max(scores2, axis=-1, keepdims=True)
    idx2 = jnp.min(jnp.where(scores2 == m2, iota, E), axis=-1, keepdims=True)
    oh2 = iota == idx2
    t = jnp.exp(m2 - m1)
    p1 = 1.0 / (1.0 + t)
    p2 = t / (1.0 + t)
    gates = p1 * oh1.astype(jnp.float32) + p2 * oh2.astype(jnp.float32)
    gate_e = jnp.sum(jnp.where(iota == e, gates, 0.0), axis=-1, keepdims=True)

    w1 = w1_ref[0]
    w2 = w2_ref[0]
    w3 = w3_ref[0]
    a = jnp.dot(xs, w1, preferred_element_type=jnp.float32)
    b = jnp.dot(xs, w2, preferred_element_type=jnp.float32)
    h = (a * jax.lax.logistic(a)) * b
    y = jnp.dot(h, w3, preferred_element_type=jnp.float32)

    @pl.when(e == 0)
    def _():
        out_ref[...] = jnp.zeros_like(out_ref)

    out_ref[...] += gate_e * y


def kernel(x, Wg, W1, W2, W3):
    B, S, D = x.shape
    E = Wg.shape[1]
    F = W1.shape[2]
    xs = x.reshape(S, D)

    out = pl.pallas_call(
        _moe_dense_kernel,
        grid=(E,),
        in_specs=[
            pl.BlockSpec((S, D), lambda e: (0, 0)),
            pl.BlockSpec((D, E), lambda e: (0, 0)),
            pl.BlockSpec((1, D, F), lambda e: (e, 0, 0)),
            pl.BlockSpec((1, D, F), lambda e: (e, 0, 0)),
            pl.BlockSpec((1, F, D), lambda e: (e, 0, 0)),
        ],
        out_specs=pl.BlockSpec((S, D), lambda e: (0, 0)),
        out_shape=jax.ShapeDtypeStruct((S, D), jnp.float32),
    )(xs, Wg, W1, W2, W3)
    return out.reshape(B, S, D)
